# baseline (speedup 1.0000x reference)
import functools

import jax
import jax.numpy as jnp
from jax import lax
from jax.experimental import pallas as pl
from jax.experimental.pallas import tpu as pltpu
from jax.experimental.pallas import tpu_sc as plsc

f32 = jnp.float32
bf16 = jnp.bfloat16

BLK = 112
NBLK_PER_W = 14
BT = 896


def _sc_gather(table, idx3, Np, w, dtype):
    info = plsc.get_sparse_core_info()
    NC, NS = info.num_cores, info.num_subcores
    NW = NC * NS
    nr = Np // BLK // NW
    mesh = plsc.VectorSubcoreMesh(core_axis_name="c", subcore_axis_name="s")

    @functools.partial(
        pl.kernel, mesh=mesh,
        out_type=jax.ShapeDtypeStruct((Np // BLK, 3, BLK, w), dtype),
        scratch_types=[
            pltpu.VMEM((2, 3, BLK), jnp.int32),
            pltpu.VMEM((3, BLK, w), dtype),
            pltpu.SemaphoreType.DMA,
            pltpu.SemaphoreType.DMA,
            pltpu.SemaphoreType.DMA,
            pltpu.SemaphoreType.DMA,
            pltpu.SemaphoreType.DMA,
            pltpu.SemaphoreType.DMA,
            pltpu.SemaphoreType.DMA,
            pltpu.SemaphoreType.DMA,
        ],
    )
    def k(table_hbm, idx_hbm, out_hbm, idx_v, rows_v,
          si0, si1, sg0, sg1, sg2, sw0, sw1, sw2):
        wid = lax.axis_index("s") * NC + lax.axis_index("c")
        g0 = wid * nr
        sis = (si0, si1)
        sgs = (sg0, sg1, sg2)
        sws = (sw0, sw1, sw2)

        pltpu.sync_copy(idx_hbm.at[:, g0], idx_v.at[0])
        pltpu.async_copy(idx_hbm.at[:, g0 + 1], idx_v.at[1], sis[1])
        for b in range(3):
            pltpu.async_copy(table_hbm.at[idx_v.at[0, b]], rows_v.at[b],
                             sgs[b])

        def one_round(r, p):
            g = g0 + r
            for b in range(3):
                pltpu.make_async_copy(table_hbm.at[idx_v.at[p, b]],
                                      rows_v.at[b], sgs[b]).wait()
                pltpu.async_copy(rows_v.at[b], out_hbm.at[g, b], sws[b])
            @pl.when(r + 2 < nr)
            def _():
                pltpu.async_copy(idx_hbm.at[:, g + 2], idx_v.at[p], sis[p])
            @pl.when(r + 1 < nr)
            def _():
                pltpu.make_async_copy(idx_hbm.at[:, g + 1], idx_v.at[1 - p],
                                      sis[1 - p]).wait()
                for b in range(3):
                    pltpu.make_async_copy(rows_v.at[b], out_hbm.at[g, b],
                                          sws[b]).wait()
                    pltpu.async_copy(table_hbm.at[idx_v.at[1 - p, b]],
                                     rows_v.at[b], sgs[b])

        def body(i, carry):
            one_round(2 * i, 0)
            one_round(2 * i + 1, 1)
            return carry

        lax.fori_loop(0, nr // 2, body, 0)
        for b in range(3):
            pltpu.make_async_copy(rows_v.at[b], out_hbm.at[g0 + nr - 1, b],
                                  sws[b]).wait()

    return k(table, idx3)


def _mmT(x, w):
    return lax.dot_general(x, w, (((1,), (1,)), ((), ())),
                           preferred_element_type=f32)


def kernel(centre, corner, normal, neighbour, sp_W2, sp_b2, sp_W1, sp_b1,
           frc_conv_w, frc_conv_b, frc_W3, frc_b3, frc_W4, frc_b4, kc_kernels,
           c1_W, c1_b, a1_W, a1_b, c2_W, c2_b, a2_W, a2_b, m2_W, m2_b,
           m3_W1, m3_b1, m3_W2, m3_b2, m3_W3, m3_b3):
    N = centre.shape[0]
    chunk = BLK * NBLK_PER_W * 32
    Np = ((N + chunk - 1) // chunk) * chunk
    NB = Np // BLK

    nbr = jnp.pad(neighbour.astype(jnp.int32), ((0, Np - N), (0, 0)))
    idx3 = nbr.T.reshape(3, NB, BLK)

    E = jnp.zeros((12, 9), f32).at[jnp.arange(12), jnp.arange(12) % 9].set(1.0)
    Mc = sum(frc_conv_w @ E[3 * w:3 * w + 6, :] for w in range(3)) / 3.0
    kflat = kc_kernels.reshape(-1, 3)
    ksq = jnp.sum(kflat ** 2, axis=-1)[None, :]
    P = ((jnp.arange(256)[:, None] // 4) == jnp.arange(64)[None, :]
         ).astype(f32) / 16.0
    hi = functools.partial(jnp.dot, precision=lax.Precision.HIGHEST)
    c1a = c1_W[:, :64]
    Wsp = hi(c1a, sp_W1)
    bias1 = (c1_b + hi(c1a, sp_b1))[None, :]
    c1fk = c1_W[:, 64:192]
    c1n = c1_W[:, 192:195]
    a1fk = a1_W[:, :128]
    a1n = a1_W[:, 128:131]
    Wf = m2_W[:, 256:768] + m2_W[:, 768:1280]
    A = m2_W[:, :256] + hi(Wf, c2_W[:, :256])
    Cm = hi(m2_W[:, 1280:], a2_W)
    B2 = hi(Wf, c2_W[:, 256:512]) + 0.25 * Cm
    C4 = 0.25 * Cm
    cvec = (m2_b + hi(Wf, c2_b) + hi(m2_W[:, 1280:], a2_b))[None, :]
    A16, B216, C416 = A.astype(bf16), B2.astype(bf16), C4.astype(bf16)

    grid = (Np // BT,)
    row_spec = lambda w: pl.BlockSpec((BT, w), lambda i: (i, 0))
    g_spec = lambda w: pl.BlockSpec((BT // BLK, 3, BLK, w),
                                    lambda i: (i, 0, 0, 0))
    full = lambda a: pl.BlockSpec(a.shape, lambda i: tuple(0 for _ in a.shape))

    def tc0(nm_r, kf_r, ksq_r, P_r, o_r):
        nm = nm_r[...]
        d2 = jnp.maximum(jnp.sum(nm * nm, axis=-1)[:, None] + ksq_r[...]
                         - 2.0 * _mmT(nm, kf_r[...]), 0.)
        phi = jnp.dot(jnp.exp(d2 * -12.5), P_r[...],
                      preferred_element_type=f32)
        o_r[...] = jnp.concatenate(
            [phi, nm, jnp.zeros((BT, 61), f32)], axis=1)

    g1t = pl.pallas_call(
        tc0, grid=grid,
        in_specs=[row_spec(3)] + [full(a) for a in (kflat, ksq, P)],
        out_specs=row_spec(128),
        out_shape=jax.ShapeDtypeStruct((N, 128), f32),
    )(normal, kflat, ksq, P)

    gph = _sc_gather(g1t, idx3, Np, 128, f32)

    def tc1(cen_r, cor_r, g1t_r, gph_r, spW2_r, spb2_r, Wsp_r, b1_r, Mc_r,
            fcb_r, W3_r, b3_r, W4_r, b4_r, c1fk_r, c1n_r,
            st_o, out1_o, nagg_o):
        h = jnp.maximum(_mmT(cen_r[...], spW2_r[...]) + spb2_r[...], 0.)
        na = _mmT(cor_r[...], Mc_r[...]) + fcb_r[...]
        frc = _mmT(jnp.maximum(_mmT(na, W3_r[...]) + b3_r[...], 0.),
                   W4_r[...]) + b4_r[...]
        me = g1t_r[...]
        kc = me[:, 0:64]
        nsum = me[:, 64:67]
        for m in range(3):
            gm = gph_r[:, m].reshape(BT, 128)
            kc = kc + gm[:, 0:64]
            nsum = nsum + gm[:, 64:67]
        st = jnp.concatenate([frc, kc], axis=1)
        st_o[...] = st
        out1_o[...] = (_mmT(h, Wsp_r[...]) + _mmT(st, c1fk_r[...])
                       + _mmT(me[:, 64:67], c1n_r[...])
                       + b1_r[...]).astype(bf16)
        nagg_o[...] = nsum * 0.25

    consts1 = (sp_W2, sp_b2[None, :], Wsp, bias1, Mc, frc_conv_b[None, :],
               frc_W3, frc_b3[None, :], frc_W4, frc_b4[None, :], c1fk, c1n)
    st, out1, nagg = pl.pallas_call(
        tc1, grid=grid,
        in_specs=[row_spec(3), row_spec(9), row_spec(128), g_spec(128)]
                 + [full(a) for a in consts1],
        out_specs=[row_spec(128), row_spec(256), row_spec(3)],
        out_shape=[jax.ShapeDtypeStruct((N, 128), f32),
                   jax.ShapeDtypeStruct((N, 256), bf16),
                   jax.ShapeDtypeStruct((N, 3), f32)],
    )(centre, corner, g1t, gph, *consts1)

    gs = _sc_gather(st, idx3, Np, 128, f32)

    MASK = -65536

    def tc2(st_r, gs_r, nagg_r, a1fk_r, a1n_r, a1b_r, out2_o):
        g = gs_r[...]
        agg = (st_r[...] + g[:, 0].reshape(BT, 128) + g[:, 1].reshape(BT, 128)
               + g[:, 2].reshape(BT, 128)) * 0.25
        o = (_mmT(agg, a1fk_r[...]) + _mmT(nagg_r[...], a1n_r[...])
             + a1b_r[...])
        lo = lax.bitcast_convert_type(
            o[:, :128].astype(bf16).astype(f32), jnp.int32)
        hi_ = lax.bitcast_convert_type(
            o[:, 128:].astype(bf16).astype(f32), jnp.int32)
        out2_o[...] = lax.shift_right_logical(lo, 16) | (hi_ & MASK)

    consts2 = (a1fk, a1n, a1_b[None, :])
    out2 = pl.pallas_call(
        tc2, grid=grid,
        in_specs=[row_spec(128), g_spec(128), row_spec(3)]
                 + [full(a) for a in consts2],
        out_specs=row_spec(128),
        out_shape=jax.ShapeDtypeStruct((N, 128), jnp.int32),
    )(st, gs, nagg, *consts2)

    go = _sc_gather(out2, idx3, Np, 128, jnp.int32)

    NLAST = Np // BT - 1

    def _unpack(x):
        lo = lax.bitcast_convert_type(lax.shift_left(x, 16), f32)
        hi_ = lax.bitcast_convert_type(x & MASK, f32)
        return lo, hi_

    def tc3(out1_r, out2_r, go_r, A_r, B2_r, C4_r, cvec_r, gmax_o):
        i = pl.program_id(0)
        s_lo, s_hi = _unpack(out2_r[...])
        g = go_r[...]
        g0lo, g0hi = _unpack(g[:, 0].reshape(BT, 128))
        g1lo, g1hi = _unpack(g[:, 1].reshape(BT, 128))
        g2lo, g2hi = _unpack(g[:, 2].reshape(BT, 128))
        y = (_mmT(out1_r[...], A_r[...])
             + _mmT(s_lo.astype(bf16), B2_r[...][:, :128])
             + _mmT(s_hi.astype(bf16), B2_r[...][:, 128:])
             + _mmT((g0lo + g1lo + g2lo).astype(bf16), C4_r[...][:, :128])
             + _mmT((g0hi + g1hi + g2hi).astype(bf16), C4_r[...][:, 128:])
             + cvec_r[...])

        @pl.when(i == 0)
        def _():
            gmax_o[...] = jnp.full((8, 1024), -jnp.inf, f32)

        @pl.when(i < NLAST)
        def _():
            m = jnp.max(y, axis=0, keepdims=True)
            gmax_o[...] = jnp.maximum(gmax_o[...],
                                      jnp.broadcast_to(m, (8, 1024)))

        @pl.when(i == NLAST)
        def _():
            rows = i * BT + lax.broadcasted_iota(jnp.int32, (BT, 1), 0)
            ym = jnp.where(rows < N, y, -jnp.inf)
            m = jnp.max(ym, axis=0, keepdims=True)
            gmax_o[...] = jnp.maximum(gmax_o[...],
                                      jnp.broadcast_to(m, (8, 1024)))

    consts3 = (A16, B216, C416, cvec)
    gmax = pl.pallas_call(
        tc3, grid=grid,
        in_specs=[row_spec(256), row_spec(128), g_spec(128)]
                 + [full(a) for a in consts3],
        out_specs=pl.BlockSpec((8, 1024), lambda i: (0, 0)),
        out_shape=jax.ShapeDtypeStruct((8, 1024), f32),
    )(out1, out2, go, *consts3)

    def tc4(g_r, W1_r, b1_r, W2_r, b2_r, W3_r, b3_r, o_r):
        h = jnp.maximum(_mmT(g_r[...], W1_r[...]) + b1_r[...], 0.)
        h = jnp.maximum(_mmT(h, W2_r[...]) + b2_r[...], 0.)
        o_r[...] = _mmT(h, W3_r[...]) + b3_r[...]

    logits8 = pl.pallas_call(
        tc4,
        out_shape=jax.ShapeDtypeStruct((8, m3_W3.shape[0]), f32),
    )(gmax, m3_W1, m3_b1[None, :], m3_W2, m3_b2[None, :], m3_W3,
      m3_b3[None, :])
    return logits8[0:1, :]


# speedup vs baseline: 1.0735x; 1.0735x over previous
import functools

import jax
import jax.numpy as jnp
from jax import lax
from jax.experimental import pallas as pl
from jax.experimental.pallas import tpu as pltpu
from jax.experimental.pallas import tpu_sc as plsc

f32 = jnp.float32
bf16 = jnp.bfloat16

BLK = 112
NBLK_PER_W = 14
BT = 896


def _sc_gather(table, idx3, Np, w, dtype):
    info = plsc.get_sparse_core_info()
    NC, NS = info.num_cores, info.num_subcores
    NW = NC * NS
    nr = Np // BLK // NW
    mesh = plsc.VectorSubcoreMesh(core_axis_name="c", subcore_axis_name="s")

    @functools.partial(
        pl.kernel, mesh=mesh,
        out_type=jax.ShapeDtypeStruct((Np // BLK, 3, BLK, w), dtype),
        scratch_types=[
            pltpu.VMEM((2, 3, BLK), jnp.int32),
            pltpu.VMEM((3, BLK, w), dtype),
            pltpu.SemaphoreType.DMA,
            pltpu.SemaphoreType.DMA,
            pltpu.SemaphoreType.DMA,
            pltpu.SemaphoreType.DMA,
            pltpu.SemaphoreType.DMA,
            pltpu.SemaphoreType.DMA,
            pltpu.SemaphoreType.DMA,
            pltpu.SemaphoreType.DMA,
        ],
    )
    def k(table_hbm, idx_hbm, out_hbm, idx_v, rows_v,
          si0, si1, sg0, sg1, sg2, sw0, sw1, sw2):
        wid = lax.axis_index("s") * NC + lax.axis_index("c")
        g0 = wid * nr
        sis = (si0, si1)
        sgs = (sg0, sg1, sg2)
        sws = (sw0, sw1, sw2)

        pltpu.sync_copy(idx_hbm.at[:, g0], idx_v.at[0])
        pltpu.async_copy(idx_hbm.at[:, g0 + 1], idx_v.at[1], sis[1])
        for b in range(3):
            pltpu.async_copy(table_hbm.at[idx_v.at[0, b]], rows_v.at[b],
                             sgs[b])

        def one_round(r, p):
            g = g0 + r
            for b in range(3):
                pltpu.make_async_copy(table_hbm.at[idx_v.at[p, b]],
                                      rows_v.at[b], sgs[b]).wait()
                pltpu.async_copy(rows_v.at[b], out_hbm.at[g, b], sws[b])
            @pl.when(r + 2 < nr)
            def _():
                pltpu.async_copy(idx_hbm.at[:, g + 2], idx_v.at[p], sis[p])
            @pl.when(r + 1 < nr)
            def _():
                pltpu.make_async_copy(idx_hbm.at[:, g + 1], idx_v.at[1 - p],
                                      sis[1 - p]).wait()
                for b in range(3):
                    pltpu.make_async_copy(rows_v.at[b], out_hbm.at[g, b],
                                          sws[b]).wait()
                    pltpu.async_copy(table_hbm.at[idx_v.at[1 - p, b]],
                                     rows_v.at[b], sgs[b])

        def body(i, carry):
            one_round(2 * i, 0)
            one_round(2 * i + 1, 1)
            return carry

        lax.fori_loop(0, nr // 2, body, 0)
        for b in range(3):
            pltpu.make_async_copy(rows_v.at[b], out_hbm.at[g0 + nr - 1, b],
                                  sws[b]).wait()

    return k(table, idx3)


def _mmT(x, w):
    return lax.dot_general(x, w, (((1,), (1,)), ((), ())),
                           preferred_element_type=f32)


def kernel(centre, corner, normal, neighbour, sp_W2, sp_b2, sp_W1, sp_b1,
           frc_conv_w, frc_conv_b, frc_W3, frc_b3, frc_W4, frc_b4, kc_kernels,
           c1_W, c1_b, a1_W, a1_b, c2_W, c2_b, a2_W, a2_b, m2_W, m2_b,
           m3_W1, m3_b1, m3_W2, m3_b2, m3_W3, m3_b3):
    N = centre.shape[0]
    chunk = BLK * NBLK_PER_W * 32
    Np = ((N + chunk - 1) // chunk) * chunk
    NB = Np // BLK

    nbr = jnp.pad(neighbour.astype(jnp.int32), ((0, Np - N), (0, 0)))
    idx3 = nbr.T.reshape(3, NB, BLK)

    E = jnp.zeros((12, 9), f32).at[jnp.arange(12), jnp.arange(12) % 9].set(1.0)
    Mc = sum(frc_conv_w @ E[3 * w:3 * w + 6, :] for w in range(3)) / 3.0
    kflat = kc_kernels.reshape(-1, 3)
    ksq = jnp.sum(kflat ** 2, axis=-1)[None, :]
    P = ((jnp.arange(256)[:, None] // 4) == jnp.arange(64)[None, :]
         ).astype(f32) / 16.0
    hi = functools.partial(jnp.dot, precision=lax.Precision.HIGHEST)
    c1a = c1_W[:, :64]
    Wsp = hi(c1a, sp_W1)
    bias1 = (c1_b + hi(c1a, sp_b1))[None, :]
    c1fk = c1_W[:, 64:192]
    c1n = c1_W[:, 192:195]
    a1fk = a1_W[:, :128]
    a1n = a1_W[:, 128:131]
    Wf = m2_W[:, 256:768] + m2_W[:, 768:1280]
    A = m2_W[:, :256] + hi(Wf, c2_W[:, :256])
    Cm = hi(m2_W[:, 1280:], a2_W)
    B2 = hi(Wf, c2_W[:, 256:512]) + 0.25 * Cm
    C4 = 0.25 * Cm
    cvec = (m2_b + hi(Wf, c2_b) + hi(m2_W[:, 1280:], a2_b))[None, :]
    Wbig = jnp.concatenate([A, B2, C4], axis=1).astype(bf16)
    rowmask = jnp.where(jnp.arange(Np) < N, 0.0,
                        -jnp.inf).astype(f32)[:, None]

    grid = (Np // BT,)
    row_spec = lambda w: pl.BlockSpec((BT, w), lambda i: (i, 0))
    g_spec = lambda w: pl.BlockSpec((BT // BLK, 3, BLK, w),
                                    lambda i: (i, 0, 0, 0))
    full = lambda a: pl.BlockSpec(a.shape, lambda i: tuple(0 for _ in a.shape))

    def tc0(nm_r, kf_r, ksq_r, P_r, o_r):
        nm = nm_r[...]
        d2 = jnp.maximum(jnp.sum(nm * nm, axis=-1)[:, None] + ksq_r[...]
                         - 2.0 * _mmT(nm, kf_r[...]), 0.)
        phi = jnp.dot(jnp.exp(d2 * -12.5), P_r[...],
                      preferred_element_type=f32)
        o_r[...] = jnp.concatenate(
            [phi, nm, jnp.zeros((BT, 61), f32)], axis=1)

    g1t = pl.pallas_call(
        tc0, grid=grid,
        in_specs=[row_spec(3)] + [full(a) for a in (kflat, ksq, P)],
        out_specs=row_spec(128),
        out_shape=jax.ShapeDtypeStruct((N, 128), f32),
    )(normal, kflat, ksq, P)

    gph = _sc_gather(g1t, idx3, Np, 128, f32)

    def tc1(cen_r, cor_r, g1t_r, gph_r, spW2_r, spb2_r, Wsp_r, b1_r, Mc_r,
            fcb_r, W3_r, b3_r, W4_r, b4_r, c1fk_r, c1n_r,
            st_o, out1_o, nagg_o):
        h = jnp.maximum(_mmT(cen_r[...], spW2_r[...]) + spb2_r[...], 0.)
        na = _mmT(cor_r[...], Mc_r[...]) + fcb_r[...]
        frc = _mmT(jnp.maximum(_mmT(na, W3_r[...]) + b3_r[...], 0.),
                   W4_r[...]) + b4_r[...]
        me = g1t_r[...]
        kc = me[:, 0:64]
        nsum = me[:, 64:67]
        for m in range(3):
            gm = gph_r[:, m].reshape(BT, 128)
            kc = kc + gm[:, 0:64]
            nsum = nsum + gm[:, 64:67]
        st = jnp.concatenate([frc, kc], axis=1)
        st_o[...] = st
        out1_o[...] = (_mmT(h, Wsp_r[...]) + _mmT(st, c1fk_r[...])
                       + _mmT(me[:, 64:67], c1n_r[...])
                       + b1_r[...]).astype(bf16)
        nagg_o[...] = nsum * 0.25

    consts1 = (sp_W2, sp_b2[None, :], Wsp, bias1, Mc, frc_conv_b[None, :],
               frc_W3, frc_b3[None, :], frc_W4, frc_b4[None, :], c1fk, c1n)
    st, out1, nagg = pl.pallas_call(
        tc1, grid=grid,
        in_specs=[row_spec(3), row_spec(9), row_spec(128), g_spec(128)]
                 + [full(a) for a in consts1],
        out_specs=[row_spec(128), row_spec(256), row_spec(3)],
        out_shape=[jax.ShapeDtypeStruct((N, 128), f32),
                   jax.ShapeDtypeStruct((N, 256), bf16),
                   jax.ShapeDtypeStruct((N, 3), f32)],
    )(centre, corner, g1t, gph, *consts1)

    gs = _sc_gather(st, idx3, Np, 128, f32)

    MASK = -65536

    def tc2(st_r, gs_r, nagg_r, a1fk_r, a1n_r, a1b_r, out2_o):
        g = gs_r[...]
        agg = (st_r[...] + g[:, 0].reshape(BT, 128) + g[:, 1].reshape(BT, 128)
               + g[:, 2].reshape(BT, 128)) * 0.25
        o = (_mmT(agg, a1fk_r[...]) + _mmT(nagg_r[...], a1n_r[...])
             + a1b_r[...])
        lo = lax.bitcast_convert_type(
            o[:, :128].astype(bf16).astype(f32), jnp.int32)
        hi_ = lax.bitcast_convert_type(
            o[:, 128:].astype(bf16).astype(f32), jnp.int32)
        out2_o[...] = lax.shift_right_logical(lo, 16) | (hi_ & MASK)

    consts2 = (a1fk, a1n, a1_b[None, :])
    out2 = pl.pallas_call(
        tc2, grid=grid,
        in_specs=[row_spec(128), g_spec(128), row_spec(3)]
                 + [full(a) for a in consts2],
        out_specs=row_spec(128),
        out_shape=jax.ShapeDtypeStruct((N, 128), jnp.int32),
    )(st, gs, nagg, *consts2)

    go = _sc_gather(out2, idx3, Np, 128, jnp.int32)

    def _unpack(x):
        lo = lax.bitcast_convert_type(lax.shift_left(x, 16), f32)
        hi_ = lax.bitcast_convert_type(x & MASK, f32)
        return lo, hi_

    def tc3(out1_r, out2_r, go_r, mask_r, W_r, cvec_r, gmax_o):
        i = pl.program_id(0)
        s_lo, s_hi = _unpack(out2_r[...])
        g = go_r[...]
        g0lo, g0hi = _unpack(g[:, 0].reshape(BT, 128))
        g1lo, g1hi = _unpack(g[:, 1].reshape(BT, 128))
        g2lo, g2hi = _unpack(g[:, 2].reshape(BT, 128))
        x = jnp.concatenate(
            [out1_r[...], s_lo.astype(bf16), s_hi.astype(bf16),
             (g0lo + g1lo + g2lo).astype(bf16),
             (g0hi + g1hi + g2hi).astype(bf16)], axis=1)
        y = _mmT(x, W_r[...]) + cvec_r[...] + mask_r[...]

        @pl.when(i == 0)
        def _():
            gmax_o[...] = jnp.full((8, 1024), -jnp.inf, f32)

        m = jnp.max(y, axis=0, keepdims=True)
        gmax_o[...] = jnp.maximum(gmax_o[...], jnp.broadcast_to(m, (8, 1024)))

    consts3 = (Wbig, cvec)
    gmax = pl.pallas_call(
        tc3, grid=grid,
        in_specs=[row_spec(256), row_spec(128), g_spec(128),
                  pl.BlockSpec((BT, 1), lambda i: (i, 0))]
                 + [full(a) for a in consts3],
        out_specs=pl.BlockSpec((8, 1024), lambda i: (0, 0)),
        out_shape=jax.ShapeDtypeStruct((8, 1024), f32),
    )(out1, out2, go, rowmask, *consts3)

    def tc4(g_r, W1_r, b1_r, W2_r, b2_r, W3_r, b3_r, o_r):
        h = jnp.maximum(_mmT(g_r[...], W1_r[...]) + b1_r[...], 0.)
        h = jnp.maximum(_mmT(h, W2_r[...]) + b2_r[...], 0.)
        o_r[...] = _mmT(h, W3_r[...]) + b3_r[...]

    logits8 = pl.pallas_call(
        tc4,
        out_shape=jax.ShapeDtypeStruct((8, m3_W3.shape[0]), f32),
    )(gmax, m3_W1, m3_b1[None, :], m3_W2, m3_b2[None, :], m3_W3,
      m3_b3[None, :])
    return logits8[0:1, :]


# speedup vs baseline: 1.1689x; 1.0888x over previous
import functools

import jax
import jax.numpy as jnp
from jax import lax
from jax.experimental import pallas as pl
from jax.experimental.pallas import tpu as pltpu
from jax.experimental.pallas import tpu_sc as plsc

f32 = jnp.float32
bf16 = jnp.bfloat16

BLK = 112
NBLK_PER_W = 14
BT = 1792


def _sc_gather(table, idx3, Np, w, dtype):
    info = plsc.get_sparse_core_info()
    NC, NS = info.num_cores, info.num_subcores
    NW = NC * NS
    nr = Np // BLK // NW
    mesh = plsc.VectorSubcoreMesh(core_axis_name="c", subcore_axis_name="s")

    @functools.partial(
        pl.kernel, mesh=mesh,
        out_type=jax.ShapeDtypeStruct((Np // BLK, 3, BLK, w), dtype),
        scratch_types=[
            pltpu.VMEM((2, 3, BLK), jnp.int32),
            pltpu.VMEM((3, BLK, w), dtype),
            pltpu.SemaphoreType.DMA,
            pltpu.SemaphoreType.DMA,
            pltpu.SemaphoreType.DMA,
            pltpu.SemaphoreType.DMA,
            pltpu.SemaphoreType.DMA,
            pltpu.SemaphoreType.DMA,
            pltpu.SemaphoreType.DMA,
            pltpu.SemaphoreType.DMA,
        ],
    )
    def k(table_hbm, idx_hbm, out_hbm, idx_v, rows_v,
          si0, si1, sg0, sg1, sg2, sw0, sw1, sw2):
        wid = lax.axis_index("s") * NC + lax.axis_index("c")
        g0 = wid * nr
        sis = (si0, si1)
        sgs = (sg0, sg1, sg2)
        sws = (sw0, sw1, sw2)

        pltpu.sync_copy(idx_hbm.at[:, g0], idx_v.at[0])
        pltpu.async_copy(idx_hbm.at[:, g0 + 1], idx_v.at[1], sis[1])
        for b in range(3):
            pltpu.async_copy(table_hbm.at[idx_v.at[0, b]], rows_v.at[b],
                             sgs[b])

        def one_round(r, p):
            g = g0 + r
            for b in range(3):
                pltpu.make_async_copy(table_hbm.at[idx_v.at[p, b]],
                                      rows_v.at[b], sgs[b]).wait()
                pltpu.async_copy(rows_v.at[b], out_hbm.at[g, b], sws[b])
            @pl.when(r + 2 < nr)
            def _():
                pltpu.async_copy(idx_hbm.at[:, g + 2], idx_v.at[p], sis[p])
            @pl.when(r + 1 < nr)
            def _():
                pltpu.make_async_copy(idx_hbm.at[:, g + 1], idx_v.at[1 - p],
                                      sis[1 - p]).wait()
                for b in range(3):
                    pltpu.make_async_copy(rows_v.at[b], out_hbm.at[g, b],
                                          sws[b]).wait()
                    pltpu.async_copy(table_hbm.at[idx_v.at[1 - p, b]],
                                     rows_v.at[b], sgs[b])

        def body(i, carry):
            one_round(2 * i, 0)
            one_round(2 * i + 1, 1)
            return carry

        lax.fori_loop(0, nr // 2, body, 0)
        for b in range(3):
            pltpu.make_async_copy(rows_v.at[b], out_hbm.at[g0 + nr - 1, b],
                                  sws[b]).wait()

    return k(table, idx3)


def _mmT(x, w):
    return lax.dot_general(x, w, (((1,), (1,)), ((), ())),
                           preferred_element_type=f32)


def kernel(centre, corner, normal, neighbour, sp_W2, sp_b2, sp_W1, sp_b1,
           frc_conv_w, frc_conv_b, frc_W3, frc_b3, frc_W4, frc_b4, kc_kernels,
           c1_W, c1_b, a1_W, a1_b, c2_W, c2_b, a2_W, a2_b, m2_W, m2_b,
           m3_W1, m3_b1, m3_W2, m3_b2, m3_W3, m3_b3):
    N = centre.shape[0]
    chunk = BLK * NBLK_PER_W * 32
    Np = ((N + chunk - 1) // chunk) * chunk
    NB = Np // BLK

    nbr = jnp.pad(neighbour.astype(jnp.int32), ((0, Np - N), (0, 0)))
    idx3 = nbr.T.reshape(3, NB, BLK)

    E = jnp.zeros((12, 9), f32).at[jnp.arange(12), jnp.arange(12) % 9].set(1.0)
    Mc = sum(frc_conv_w @ E[3 * w:3 * w + 6, :] for w in range(3)) / 3.0
    kflat = kc_kernels.reshape(-1, 3)
    ksq = jnp.sum(kflat ** 2, axis=-1)[None, :]
    P = ((jnp.arange(256)[:, None] // 4) == jnp.arange(64)[None, :]
         ).astype(f32) / 16.0
    hi = functools.partial(jnp.dot, precision=lax.Precision.HIGHEST)
    c1a = c1_W[:, :64]
    Wsp = hi(c1a, sp_W1)
    bias1 = (c1_b + hi(c1a, sp_b1))[None, :]
    c1fk = c1_W[:, 64:192]
    c1n = c1_W[:, 192:195]
    a1fk = a1_W[:, :128]
    a1n = a1_W[:, 128:131]
    Wf = m2_W[:, 256:768] + m2_W[:, 768:1280]
    A = m2_W[:, :256] + hi(Wf, c2_W[:, :256])
    Cm = hi(m2_W[:, 1280:], a2_W)
    B2 = hi(Wf, c2_W[:, 256:512]) + 0.25 * Cm
    C4 = 0.25 * Cm
    cvec = (m2_b + hi(Wf, c2_b) + hi(m2_W[:, 1280:], a2_b))[None, :]
    Wbig = jnp.concatenate([A, B2, C4], axis=1).astype(bf16)
    rowmask = jnp.where(jnp.arange(Np) < N, 0.0,
                        -jnp.inf).astype(f32)[:, None]

    grid = (Np // BT,)
    row_spec = lambda w: pl.BlockSpec((BT, w), lambda i: (i, 0))
    g_spec = lambda w: pl.BlockSpec((BT // BLK, 3, BLK, w),
                                    lambda i: (i, 0, 0, 0))
    full = lambda a: pl.BlockSpec(a.shape, lambda i: tuple(0 for _ in a.shape))

    def tc0(nm_r, kf_r, ksq_r, P_r, o_r):
        nm = nm_r[...]
        d2 = jnp.maximum(jnp.sum(nm * nm, axis=-1)[:, None] + ksq_r[...]
                         - 2.0 * _mmT(nm, kf_r[...]), 0.)
        phi = jnp.dot(jnp.exp(d2 * -12.5), P_r[...],
                      preferred_element_type=f32)
        o_r[...] = jnp.concatenate(
            [phi, nm, jnp.zeros((BT, 61), f32)], axis=1)

    g1t = pl.pallas_call(
        tc0, grid=grid,
        in_specs=[row_spec(3)] + [full(a) for a in (kflat, ksq, P)],
        out_specs=row_spec(128),
        out_shape=jax.ShapeDtypeStruct((N, 128), f32),
    )(normal, kflat, ksq, P)

    gph = _sc_gather(g1t, idx3, Np, 128, f32)

    def tc1(cen_r, cor_r, g1t_r, gph_r, spW2_r, spb2_r, Wsp_r, b1_r, Mc_r,
            fcb_r, W3_r, b3_r, W4_r, b4_r, c1fk_r, c1n_r,
            st_o, out1_o, nagg_o):
        h = jnp.maximum(_mmT(cen_r[...], spW2_r[...]) + spb2_r[...], 0.)
        na = _mmT(cor_r[...], Mc_r[...]) + fcb_r[...]
        frc = _mmT(jnp.maximum(_mmT(na, W3_r[...]) + b3_r[...], 0.),
                   W4_r[...]) + b4_r[...]
        me = g1t_r[...]
        kc = me[:, 0:64]
        nsum = me[:, 64:67]
        for m in range(3):
            gm = gph_r[:, m].reshape(BT, 128)
            kc = kc + gm[:, 0:64]
            nsum = nsum + gm[:, 64:67]
        st = jnp.concatenate([frc, kc], axis=1)
        st_o[...] = st
        out1_o[...] = (_mmT(h, Wsp_r[...]) + _mmT(st, c1fk_r[...])
                       + _mmT(me[:, 64:67], c1n_r[...])
                       + b1_r[...]).astype(bf16)
        nagg_o[...] = nsum * 0.25

    consts1 = (sp_W2, sp_b2[None, :], Wsp, bias1, Mc, frc_conv_b[None, :],
               frc_W3, frc_b3[None, :], frc_W4, frc_b4[None, :], c1fk, c1n)
    st, out1, nagg = pl.pallas_call(
        tc1, grid=grid,
        in_specs=[row_spec(3), row_spec(9), row_spec(128), g_spec(128)]
                 + [full(a) for a in consts1],
        out_specs=[row_spec(128), row_spec(256), row_spec(3)],
        out_shape=[jax.ShapeDtypeStruct((N, 128), f32),
                   jax.ShapeDtypeStruct((N, 256), bf16),
                   jax.ShapeDtypeStruct((N, 3), f32)],
    )(centre, corner, g1t, gph, *consts1)

    gs = _sc_gather(st, idx3, Np, 128, f32)

    MASK = -65536

    def tc2(st_r, gs_r, nagg_r, a1fk_r, a1n_r, a1b_r, out2_o):
        g = gs_r[...]
        agg = (st_r[...] + g[:, 0].reshape(BT, 128) + g[:, 1].reshape(BT, 128)
               + g[:, 2].reshape(BT, 128)) * 0.25
        o = (_mmT(agg, a1fk_r[...]) + _mmT(nagg_r[...], a1n_r[...])
             + a1b_r[...])
        lo = lax.bitcast_convert_type(
            o[:, :128].astype(bf16).astype(f32), jnp.int32)
        hi_ = lax.bitcast_convert_type(
            o[:, 128:].astype(bf16).astype(f32), jnp.int32)
        out2_o[...] = lax.shift_right_logical(lo, 16) | (hi_ & MASK)

    consts2 = (a1fk, a1n, a1_b[None, :])
    out2 = pl.pallas_call(
        tc2, grid=grid,
        in_specs=[row_spec(128), g_spec(128), row_spec(3)]
                 + [full(a) for a in consts2],
        out_specs=row_spec(128),
        out_shape=jax.ShapeDtypeStruct((N, 128), jnp.int32),
    )(st, gs, nagg, *consts2)

    go = _sc_gather(out2, idx3, Np, 128, jnp.int32)

    def _unpack(x):
        lo = lax.bitcast_convert_type(lax.shift_left(x, 16), f32)
        hi_ = lax.bitcast_convert_type(x & MASK, f32)
        return lo, hi_

    def tc3(out1_r, out2_r, go_r, mask_r, W_r, cvec_r, gmax_o):
        i = pl.program_id(0)
        s_lo, s_hi = _unpack(out2_r[...])
        g = go_r[...]
        g0lo, g0hi = _unpack(g[:, 0].reshape(BT, 128))
        g1lo, g1hi = _unpack(g[:, 1].reshape(BT, 128))
        g2lo, g2hi = _unpack(g[:, 2].reshape(BT, 128))
        x = jnp.concatenate(
            [out1_r[...], s_lo.astype(bf16), s_hi.astype(bf16),
             (g0lo + g1lo + g2lo).astype(bf16),
             (g0hi + g1hi + g2hi).astype(bf16)], axis=1)
        y = _mmT(x, W_r[...]) + cvec_r[...] + mask_r[...]

        @pl.when(i == 0)
        def _():
            gmax_o[...] = jnp.full((8, 1024), -jnp.inf, f32)

        m = jnp.max(y, axis=0, keepdims=True)
        gmax_o[...] = jnp.maximum(gmax_o[...], jnp.broadcast_to(m, (8, 1024)))

    consts3 = (Wbig, cvec)
    gmax = pl.pallas_call(
        tc3, grid=grid,
        in_specs=[row_spec(256), row_spec(128), g_spec(128),
                  pl.BlockSpec((BT, 1), lambda i: (i, 0))]
                 + [full(a) for a in consts3],
        out_specs=pl.BlockSpec((8, 1024), lambda i: (0, 0)),
        out_shape=jax.ShapeDtypeStruct((8, 1024), f32),
    )(out1, out2, go, rowmask, *consts3)

    def tc4(g_r, W1_r, b1_r, W2_r, b2_r, W3_r, b3_r, o_r):
        h = jnp.maximum(_mmT(g_r[...], W1_r[...]) + b1_r[...], 0.)
        h = jnp.maximum(_mmT(h, W2_r[...]) + b2_r[...], 0.)
        o_r[...] = _mmT(h, W3_r[...]) + b3_r[...]

    logits8 = pl.pallas_call(
        tc4,
        out_shape=jax.ShapeDtypeStruct((8, m3_W3.shape[0]), f32),
    )(gmax, m3_W1, m3_b1[None, :], m3_W2, m3_b2[None, :], m3_W3,
      m3_b3[None, :])
    return logits8[0:1, :]


# speedup vs baseline: 1.1996x; 1.0263x over previous
import functools

import jax
import jax.numpy as jnp
from jax import lax
from jax.experimental import pallas as pl
from jax.experimental.pallas import tpu as pltpu
from jax.experimental.pallas import tpu_sc as plsc

f32 = jnp.float32
bf16 = jnp.bfloat16

BLK = 112
NBLK_PER_W = 14
BT = 3584


def _sc_gather(table, idx3, Np, w, dtype):
    info = plsc.get_sparse_core_info()
    NC, NS = info.num_cores, info.num_subcores
    NW = NC * NS
    nr = Np // BLK // NW
    mesh = plsc.VectorSubcoreMesh(core_axis_name="c", subcore_axis_name="s")

    @functools.partial(
        pl.kernel, mesh=mesh,
        out_type=jax.ShapeDtypeStruct((Np // BLK, 3, BLK, w), dtype),
        scratch_types=[
            pltpu.VMEM((2, 3, BLK), jnp.int32),
            pltpu.VMEM((3, BLK, w), dtype),
            pltpu.SemaphoreType.DMA,
            pltpu.SemaphoreType.DMA,
            pltpu.SemaphoreType.DMA,
            pltpu.SemaphoreType.DMA,
            pltpu.SemaphoreType.DMA,
            pltpu.SemaphoreType.DMA,
            pltpu.SemaphoreType.DMA,
            pltpu.SemaphoreType.DMA,
        ],
    )
    def k(table_hbm, idx_hbm, out_hbm, idx_v, rows_v,
          si0, si1, sg0, sg1, sg2, sw0, sw1, sw2):
        wid = lax.axis_index("s") * NC + lax.axis_index("c")
        g0 = wid * nr
        sis = (si0, si1)
        sgs = (sg0, sg1, sg2)
        sws = (sw0, sw1, sw2)

        pltpu.sync_copy(idx_hbm.at[:, g0], idx_v.at[0])
        pltpu.async_copy(idx_hbm.at[:, g0 + 1], idx_v.at[1], sis[1])
        for b in range(3):
            pltpu.async_copy(table_hbm.at[idx_v.at[0, b]], rows_v.at[b],
                             sgs[b])

        def one_round(r, p):
            g = g0 + r
            for b in range(3):
                pltpu.make_async_copy(table_hbm.at[idx_v.at[p, b]],
                                      rows_v.at[b], sgs[b]).wait()
                pltpu.async_copy(rows_v.at[b], out_hbm.at[g, b], sws[b])
            @pl.when(r + 2 < nr)
            def _():
                pltpu.async_copy(idx_hbm.at[:, g + 2], idx_v.at[p], sis[p])
            @pl.when(r + 1 < nr)
            def _():
                pltpu.make_async_copy(idx_hbm.at[:, g + 1], idx_v.at[1 - p],
                                      sis[1 - p]).wait()
                for b in range(3):
                    pltpu.make_async_copy(rows_v.at[b], out_hbm.at[g, b],
                                          sws[b]).wait()
                    pltpu.async_copy(table_hbm.at[idx_v.at[1 - p, b]],
                                     rows_v.at[b], sgs[b])

        def body(i, carry):
            one_round(2 * i, 0)
            one_round(2 * i + 1, 1)
            return carry

        lax.fori_loop(0, nr // 2, body, 0)
        for b in range(3):
            pltpu.make_async_copy(rows_v.at[b], out_hbm.at[g0 + nr - 1, b],
                                  sws[b]).wait()

    return k(table, idx3)


def _mmT(x, w):
    return lax.dot_general(x, w, (((1,), (1,)), ((), ())),
                           preferred_element_type=f32)


def kernel(centre, corner, normal, neighbour, sp_W2, sp_b2, sp_W1, sp_b1,
           frc_conv_w, frc_conv_b, frc_W3, frc_b3, frc_W4, frc_b4, kc_kernels,
           c1_W, c1_b, a1_W, a1_b, c2_W, c2_b, a2_W, a2_b, m2_W, m2_b,
           m3_W1, m3_b1, m3_W2, m3_b2, m3_W3, m3_b3):
    N = centre.shape[0]
    chunk = BLK * NBLK_PER_W * 32
    Np = ((N + chunk - 1) // chunk) * chunk
    NB = Np // BLK

    nbr = jnp.pad(neighbour.astype(jnp.int32), ((0, Np - N), (0, 0)))
    idx3 = nbr.T.reshape(3, NB, BLK)

    E = jnp.zeros((12, 9), f32).at[jnp.arange(12), jnp.arange(12) % 9].set(1.0)
    Mc = sum(frc_conv_w @ E[3 * w:3 * w + 6, :] for w in range(3)) / 3.0
    kflat = kc_kernels.reshape(-1, 3)
    ksq = jnp.sum(kflat ** 2, axis=-1)[None, :]
    P = ((jnp.arange(256)[:, None] // 4) == jnp.arange(64)[None, :]
         ).astype(f32) / 16.0
    hi = functools.partial(jnp.dot, precision=lax.Precision.HIGHEST)
    c1a = c1_W[:, :64]
    Wsp = hi(c1a, sp_W1)
    bias1 = (c1_b + hi(c1a, sp_b1))[None, :]
    c1fk = c1_W[:, 64:192]
    c1n = c1_W[:, 192:195]
    a1fk = a1_W[:, :128]
    a1n = a1_W[:, 128:131]
    Wf = m2_W[:, 256:768] + m2_W[:, 768:1280]
    A = m2_W[:, :256] + hi(Wf, c2_W[:, :256])
    Cm = hi(m2_W[:, 1280:], a2_W)
    B2 = hi(Wf, c2_W[:, 256:512]) + 0.25 * Cm
    C4 = 0.25 * Cm
    cvec = (m2_b + hi(Wf, c2_b) + hi(m2_W[:, 1280:], a2_b))[None, :]
    Wbig = jnp.concatenate([A, B2, C4], axis=1).astype(bf16)
    rowmask = jnp.where(jnp.arange(Np) < N, 0.0,
                        -jnp.inf).astype(f32)[:, None]

    grid = (Np // BT,)
    row_spec = lambda w: pl.BlockSpec((BT, w), lambda i: (i, 0))
    g_spec = lambda w: pl.BlockSpec((BT // BLK, 3, BLK, w),
                                    lambda i: (i, 0, 0, 0))
    full = lambda a: pl.BlockSpec(a.shape, lambda i: tuple(0 for _ in a.shape))

    def tc0(nm_r, kf_r, ksq_r, P_r, o_r):
        nm = nm_r[...]
        d2 = jnp.maximum(jnp.sum(nm * nm, axis=-1)[:, None] + ksq_r[...]
                         - 2.0 * _mmT(nm, kf_r[...]), 0.)
        phi = jnp.dot(jnp.exp(d2 * -12.5), P_r[...],
                      preferred_element_type=f32)
        o_r[...] = jnp.concatenate(
            [phi, nm, jnp.zeros((BT, 61), f32)], axis=1)

    g1t = pl.pallas_call(
        tc0, grid=grid,
        in_specs=[row_spec(3)] + [full(a) for a in (kflat, ksq, P)],
        out_specs=row_spec(128),
        out_shape=jax.ShapeDtypeStruct((N, 128), f32),
    )(normal, kflat, ksq, P)

    gph = _sc_gather(g1t, idx3, Np, 128, f32)

    def tc1(cen_r, cor_r, g1t_r, gph_r, spW2_r, spb2_r, Wsp_r, b1_r, Mc_r,
            fcb_r, W3_r, b3_r, W4_r, b4_r, c1fk_r, c1n_r,
            st_o, out1_o, nagg_o):
        h = jnp.maximum(_mmT(cen_r[...], spW2_r[...]) + spb2_r[...], 0.)
        na = _mmT(cor_r[...], Mc_r[...]) + fcb_r[...]
        frc = _mmT(jnp.maximum(_mmT(na, W3_r[...]) + b3_r[...], 0.),
                   W4_r[...]) + b4_r[...]
        me = g1t_r[...]
        kc = me[:, 0:64]
        nsum = me[:, 64:67]
        for m in range(3):
            gm = gph_r[:, m].reshape(BT, 128)
            kc = kc + gm[:, 0:64]
            nsum = nsum + gm[:, 64:67]
        st = jnp.concatenate([frc, kc], axis=1)
        st_o[...] = st
        out1_o[...] = (_mmT(h, Wsp_r[...]) + _mmT(st, c1fk_r[...])
                       + _mmT(me[:, 64:67], c1n_r[...])
                       + b1_r[...]).astype(bf16)
        nagg_o[...] = nsum * 0.25

    consts1 = (sp_W2, sp_b2[None, :], Wsp, bias1, Mc, frc_conv_b[None, :],
               frc_W3, frc_b3[None, :], frc_W4, frc_b4[None, :], c1fk, c1n)
    st, out1, nagg = pl.pallas_call(
        tc1, grid=grid,
        in_specs=[row_spec(3), row_spec(9), row_spec(128), g_spec(128)]
                 + [full(a) for a in consts1],
        out_specs=[row_spec(128), row_spec(256), row_spec(3)],
        out_shape=[jax.ShapeDtypeStruct((N, 128), f32),
                   jax.ShapeDtypeStruct((N, 256), bf16),
                   jax.ShapeDtypeStruct((N, 3), f32)],
    )(centre, corner, g1t, gph, *consts1)

    gs = _sc_gather(st, idx3, Np, 128, f32)

    MASK = -65536

    def tc2(st_r, gs_r, nagg_r, a1fk_r, a1n_r, a1b_r, out2_o):
        g = gs_r[...]
        agg = (st_r[...] + g[:, 0].reshape(BT, 128) + g[:, 1].reshape(BT, 128)
               + g[:, 2].reshape(BT, 128)) * 0.25
        o = (_mmT(agg, a1fk_r[...]) + _mmT(nagg_r[...], a1n_r[...])
             + a1b_r[...])
        lo = lax.bitcast_convert_type(
            o[:, :128].astype(bf16).astype(f32), jnp.int32)
        hi_ = lax.bitcast_convert_type(
            o[:, 128:].astype(bf16).astype(f32), jnp.int32)
        out2_o[...] = lax.shift_right_logical(lo, 16) | (hi_ & MASK)

    consts2 = (a1fk, a1n, a1_b[None, :])
    out2 = pl.pallas_call(
        tc2, grid=grid,
        in_specs=[row_spec(128), g_spec(128), row_spec(3)]
                 + [full(a) for a in consts2],
        out_specs=row_spec(128),
        out_shape=jax.ShapeDtypeStruct((N, 128), jnp.int32),
    )(st, gs, nagg, *consts2)

    go = _sc_gather(out2, idx3, Np, 128, jnp.int32)

    def _unpack(x):
        lo = lax.bitcast_convert_type(lax.shift_left(x, 16), f32)
        hi_ = lax.bitcast_convert_type(x & MASK, f32)
        return lo, hi_

    def tc3(out1_r, out2_r, go_r, mask_r, W_r, cvec_r, gmax_o):
        i = pl.program_id(0)
        s_lo, s_hi = _unpack(out2_r[...])
        g = go_r[...]
        g0lo, g0hi = _unpack(g[:, 0].reshape(BT, 128))
        g1lo, g1hi = _unpack(g[:, 1].reshape(BT, 128))
        g2lo, g2hi = _unpack(g[:, 2].reshape(BT, 128))
        x = jnp.concatenate(
            [out1_r[...], s_lo.astype(bf16), s_hi.astype(bf16),
             (g0lo + g1lo + g2lo).astype(bf16),
             (g0hi + g1hi + g2hi).astype(bf16)], axis=1)
        y = _mmT(x, W_r[...]) + cvec_r[...] + mask_r[...]

        @pl.when(i == 0)
        def _():
            gmax_o[...] = jnp.full((8, 1024), -jnp.inf, f32)

        m = jnp.max(y, axis=0, keepdims=True)
        gmax_o[...] = jnp.maximum(gmax_o[...], jnp.broadcast_to(m, (8, 1024)))

    consts3 = (Wbig, cvec)
    gmax = pl.pallas_call(
        tc3, grid=grid,
        in_specs=[row_spec(256), row_spec(128), g_spec(128),
                  pl.BlockSpec((BT, 1), lambda i: (i, 0))]
                 + [full(a) for a in consts3],
        out_specs=pl.BlockSpec((8, 1024), lambda i: (0, 0)),
        out_shape=jax.ShapeDtypeStruct((8, 1024), f32),
    )(out1, out2, go, rowmask, *consts3)

    def tc4(g_r, W1_r, b1_r, W2_r, b2_r, W3_r, b3_r, o_r):
        h = jnp.maximum(_mmT(g_r[...], W1_r[...]) + b1_r[...], 0.)
        h = jnp.maximum(_mmT(h, W2_r[...]) + b2_r[...], 0.)
        o_r[...] = _mmT(h, W3_r[...]) + b3_r[...]

    logits8 = pl.pallas_call(
        tc4,
        out_shape=jax.ShapeDtypeStruct((8, m3_W3.shape[0]), f32),
    )(gmax, m3_W1, m3_b1[None, :], m3_W2, m3_b2[None, :], m3_W3,
      m3_b3[None, :])
    return logits8[0:1, :]
